# baseline (speedup 1.0000x reference)
import jax
import jax.numpy as jnp
from jax.experimental import pallas as pl
from jax.experimental.pallas import tpu as pltpu

_NUM_EXPERTS = 64
_TOP_K = 8
_MODEL_DIM = 2048
_BLOCK = 2048


def _compute_block(x, w, probs_view, weights_view, idx_view):
    logits = jax.lax.dot_general(
        w, x, (((1,), (1,)), ((), ())), preferred_element_type=jnp.float32
    )
    m = jnp.max(logits, axis=0, keepdims=True)
    e = jnp.exp(logits - m)
    s = jnp.sum(e, axis=0, keepdims=True)
    probs = e / s
    probs_view[...] = probs.T

    B = probs.shape[1]
    expert = jax.lax.broadcasted_iota(jnp.int32, (_NUM_EXPERTS, B), 0)
    pm = probs
    vals = []
    idxs = []
    for _ in range(_TOP_K):
        mj = jnp.max(pm, axis=0, keepdims=True)
        eq = pm == mj
        ij = jnp.min(jnp.where(eq, expert, _NUM_EXPERTS), axis=0,
                     keepdims=True)
        vals.append(mj)
        idxs.append(ij)
        pm = jnp.where(expert == ij, -jnp.inf, pm)
    v = jnp.concatenate(vals, axis=0)
    i = jnp.concatenate(idxs, axis=0)
    v = v / jnp.sum(v, axis=0, keepdims=True)
    weights_view[...] = v.T
    idx_view[...] = i.T


def _outer(x_hbm, w_ref, probs_hbm, weights_hbm, idx_hbm,
           xbuf, pbuf, vbuf, ibuf, in_sems, out_sems):
    nblk = x_hbm.shape[0] // _BLOCK
    w = w_ref[...]

    def in_copy(i, slot):
        return pltpu.make_async_copy(
            x_hbm.at[pl.ds(i * _BLOCK, _BLOCK), :], xbuf.at[slot],
            in_sems.at[slot])

    def out_copies(i, slot):
        return (
            pltpu.make_async_copy(
                pbuf.at[slot], probs_hbm.at[pl.ds(i * _BLOCK, _BLOCK), :],
                out_sems.at[slot, 0]),
            pltpu.make_async_copy(
                vbuf.at[slot], weights_hbm.at[pl.ds(i * _BLOCK, _BLOCK), :],
                out_sems.at[slot, 1]),
            pltpu.make_async_copy(
                ibuf.at[slot], idx_hbm.at[pl.ds(i * _BLOCK, _BLOCK), :],
                out_sems.at[slot, 2]),
        )

    in_copy(0, 0).start()
    for i in range(nblk):
        slot = i % 2
        if i + 1 < nblk:
            in_copy(i + 1, (i + 1) % 2).start()
        in_copy(i, slot).wait()
        if i >= 2:
            for c in out_copies(i - 2, slot):
                c.wait()
        _compute_block(xbuf[slot], w, pbuf.at[slot], vbuf.at[slot],
                       ibuf.at[slot])
        for c in out_copies(i, slot):
            c.start()
    for i in (nblk - 2, nblk - 1):
        for c in out_copies(i, i % 2):
            c.wait()


def kernel(hidden_states, weight):
    x = hidden_states.reshape(-1, _MODEL_DIM)
    T = x.shape[0]
    probs, weights, idxs = pl.pallas_call(
        _outer,
        in_specs=[
            pl.BlockSpec(memory_space=pltpu.MemorySpace.HBM),
            pl.BlockSpec(memory_space=pltpu.MemorySpace.VMEM),
        ],
        out_specs=[
            pl.BlockSpec(memory_space=pltpu.MemorySpace.HBM),
            pl.BlockSpec(memory_space=pltpu.MemorySpace.HBM),
            pl.BlockSpec(memory_space=pltpu.MemorySpace.HBM),
        ],
        out_shape=[
            jax.ShapeDtypeStruct((T, _NUM_EXPERTS), jnp.float32),
            jax.ShapeDtypeStruct((T, _TOP_K), jnp.float32),
            jax.ShapeDtypeStruct((T, _TOP_K), jnp.int32),
        ],
        scratch_shapes=[
            pltpu.VMEM((2, _BLOCK, _MODEL_DIM), jnp.float32),
            pltpu.VMEM((2, _BLOCK, _NUM_EXPERTS), jnp.float32),
            pltpu.VMEM((2, _BLOCK, _TOP_K), jnp.float32),
            pltpu.VMEM((2, _BLOCK, _TOP_K), jnp.int32),
            pltpu.SemaphoreType.DMA((2,)),
            pltpu.SemaphoreType.DMA((2, 3)),
        ],
    )(x, weight)
    return (probs, weights, idxs)
